# baseline (speedup 1.0000x reference)
import functools

import jax
import jax.numpy as jnp
from jax import lax
from jax.experimental import pallas as pl
from jax.experimental.pallas import tpu as pltpu
from jax.experimental.pallas import tpu_sc as plsc

N_NODES = 10000
D = 128
E = 320000
NC = 2
NS = 16
L = 16
NW = NC * NS
CH = 112
NCHUNK = 90
K0 = 174
K1 = 2 * NCHUNK - K0
E_PAD = NW * NCHUNK * CH
N_PAD = 10112
RPT = N_PAD // NS

_mesh = plsc.VectorSubcoreMesh(core_axis_name="c", subcore_axis_name="s")


@functools.partial(
    pl.kernel,
    out_type=jax.ShapeDtypeStruct((NC, N_PAD, D), jnp.float32),
    mesh=_mesh,
    scratch_types=[
        [pltpu.VMEM((CH, D), jnp.float32) for _ in range(3)],
        pltpu.VMEM((6, CH), jnp.int32),
        pltpu.VMEM((6, CH), jnp.int32),
        pltpu.VMEM((6, CH), jnp.float32),
        pltpu.VMEM_SHARED((N_PAD, D), jnp.float32),
        [pltpu.SemaphoreType.DMA for _ in range(3)],
        [pltpu.SemaphoreType.DMA for _ in range(3)],
        [pltpu.SemaphoreType.DMA for _ in range(6)],
    ],
)
def _spmm(support, src1, dst1, w1, out,
          gbuf, srcb, dstb, wb, acc, gsem, ssem, isem):
    c = lax.axis_index("c")
    s = lax.axis_index("s")
    k_chunks = jnp.where(c == 0, K0, K1)
    gb = k_chunks // 6
    chunk_base = jnp.where(c == 0, s * K0, NS * K0 + s * K1)
    edge_base = chunk_base * CH

    zeros = jnp.zeros((L,), jnp.float32)

    def _zero_row(e, carry):
        for j in range(D // L):
            gbuf[2][e, pl.ds(j * L, L)] = zeros
        return carry

    lax.fori_loop(0, CH, _zero_row, 0)
    base_row = s * RPT
    for r in range(RPT // CH):
        pltpu.sync_copy(gbuf[2], acc.at[pl.ds(base_row + r * CH, CH)])
    pltpu.sync_copy(gbuf[2].at[pl.ds(0, RPT % CH)],
                    acc.at[pl.ds(base_row + (RPT // CH) * CH, RPT % CH)])
    plsc.subcore_barrier()

    def _fire_idx(j, slot):
        off = edge_base + j * CH
        pltpu.async_copy(src1.at[pl.ds(off, CH)], srcb.at[slot], isem[slot])
        pltpu.async_copy(dst1.at[pl.ds(off, CH)], dstb.at[slot], isem[slot])
        pltpu.async_copy(w1.at[pl.ds(off, CH)], wb.at[slot], isem[slot])

    def _wait_idx(slot):
        pltpu.make_async_copy(src1.at[pl.ds(0, CH)], srcb.at[slot],
                              isem[slot]).wait()
        pltpu.make_async_copy(dst1.at[pl.ds(0, CH)], dstb.at[slot],
                              isem[slot]).wait()
        pltpu.make_async_copy(w1.at[pl.ds(0, CH)], wb.at[slot],
                              isem[slot]).wait()

    for j in range(5):
        _fire_idx(j, j)
    for b in range(2):
        _wait_idx(b)
        pltpu.async_copy(support.at[srcb.at[b]], gbuf[b], gsem[b])

    def _outer(g, carry):
        for b6 in range(6):
            i = g * 6 + b6
            b3 = b6 % 3

            pltpu.make_async_copy(support.at[srcb.at[b6]], gbuf[b3],
                                  gsem[b3]).wait()

            def _scale_group(grp, cc):
                wg = wb[b6, pl.ds(grp * L, L)]
                base = grp * L
                for t in range(L):
                    wvec = jnp.full((L,), wg[t], jnp.float32)
                    for j in range(D // L):
                        sl = pl.ds(j * L, L)
                        gbuf[b3][base + t, sl] = gbuf[b3][base + t, sl] * wvec
                return cc

            lax.fori_loop(0, CH // L, _scale_group, 0)

            pltpu.sync_copy(gbuf[b3], acc.at[dstb.at[b6]], add=True)

            def _refill_idx():
                _fire_idx(i + 5, (b6 + 5) % 6)

            if b6 == 0:
                _refill_idx()
            else:
                @pl.when(g < gb - 1)
                def _():
                    _refill_idx()

            def _fire_gather():
                _wait_idx((b6 + 2) % 6)
                pltpu.async_copy(support.at[srcb.at[(b6 + 2) % 6]],
                                 gbuf[(b6 + 2) % 3], gsem[(b6 + 2) % 3])

            if b6 <= 3:
                _fire_gather()
            else:
                @pl.when(g < gb - 1)
                def _():
                    _fire_gather()
        return carry

    lax.fori_loop(0, gb, _outer, 0)

    plsc.subcore_barrier()

    pltpu.sync_copy(acc.at[pl.ds(s * RPT, RPT)],
                    out.at[c, pl.ds(s * RPT, RPT)])


def _mm_body(x_ref, w_ref, o_ref):
    o_ref[...] = jnp.dot(x_ref[...], w_ref[...],
                         preferred_element_type=jnp.float32)


def _combine_body(p_ref, b_ref, o_ref):
    o_ref[...] = (p_ref[0, :N_NODES, :] + p_ref[1, :N_NODES, :]
                  + b_ref[...])


def kernel(x, edge_index, edge_weight, W, b):
    support = pl.pallas_call(
        _mm_body,
        out_shape=jax.ShapeDtypeStruct((N_NODES, D), jnp.float32),
    )(x, W)

    pad = E_PAD - E
    src1 = jnp.concatenate(
        [edge_index[1].astype(jnp.int32), jnp.zeros((pad,), jnp.int32)])
    dst1 = jnp.concatenate(
        [edge_index[0].astype(jnp.int32), jnp.zeros((pad,), jnp.int32)])
    w1 = jnp.concatenate(
        [edge_weight, jnp.zeros((pad,), jnp.float32)])

    partials = _spmm(support, src1, dst1, w1)

    return pl.pallas_call(
        _combine_body,
        out_shape=jax.ShapeDtypeStruct((N_NODES, D), jnp.float32),
    )(partials, b)


# speedup vs baseline: 1.2274x; 1.2274x over previous
import functools

import jax
import jax.numpy as jnp
from jax import lax
from jax.experimental import pallas as pl
from jax.experimental.pallas import tpu as pltpu
from jax.experimental.pallas import tpu_sc as plsc

N_NODES = 10000
D = 128
E = 320000
NC = 2
NS = 16
L = 16
NW = NC * NS
CH = 112
NCHUNK = 90
K0 = 132
K1 = 2 * NCHUNK - K0
E_PAD = NW * NCHUNK * CH
N_PAD = 10112
RPT = N_PAD // NS

_mesh = plsc.VectorSubcoreMesh(core_axis_name="c", subcore_axis_name="s")


@functools.partial(
    pl.kernel,
    out_type=jax.ShapeDtypeStruct((NC, N_PAD, D), jnp.float32),
    mesh=_mesh,
    scratch_types=[
        [pltpu.VMEM((CH, D), jnp.float32) for _ in range(3)],
        pltpu.VMEM((6, CH), jnp.int32),
        pltpu.VMEM((6, CH), jnp.int32),
        pltpu.VMEM((6, CH), jnp.float32),
        pltpu.VMEM_SHARED((N_PAD, D), jnp.float32),
        [pltpu.SemaphoreType.DMA for _ in range(3)],
        [pltpu.SemaphoreType.DMA for _ in range(3)],
        [pltpu.SemaphoreType.DMA for _ in range(6)],
    ],
)
def _spmm(support, src1, dst1, w1, out,
          gbuf, srcb, dstb, wb, acc, gsem, ssem, isem):
    c = lax.axis_index("c")
    s = lax.axis_index("s")
    k_chunks = jnp.where(c == 0, K0, K1)
    gb = k_chunks // 6
    chunk_base = jnp.where(c == 0, s * K0, NS * K0 + s * K1)
    edge_base = chunk_base * CH

    zeros = jnp.zeros((L,), jnp.float32)

    def _zero_row(e, carry):
        for j in range(D // L):
            gbuf[2][e, pl.ds(j * L, L)] = zeros
        return carry

    lax.fori_loop(0, CH, _zero_row, 0)
    base_row = s * RPT
    for r in range(RPT // CH):
        pltpu.sync_copy(gbuf[2], acc.at[pl.ds(base_row + r * CH, CH)])
    pltpu.sync_copy(gbuf[2].at[pl.ds(0, RPT % CH)],
                    acc.at[pl.ds(base_row + (RPT // CH) * CH, RPT % CH)])
    plsc.subcore_barrier()

    def _fire_idx(j, slot):
        off = edge_base + j * CH
        pltpu.async_copy(src1.at[pl.ds(off, CH)], srcb.at[slot], isem[slot])
        pltpu.async_copy(dst1.at[pl.ds(off, CH)], dstb.at[slot], isem[slot])
        pltpu.async_copy(w1.at[pl.ds(off, CH)], wb.at[slot], isem[slot])

    def _wait_idx(slot):
        pltpu.make_async_copy(src1.at[pl.ds(0, CH)], srcb.at[slot],
                              isem[slot]).wait()
        pltpu.make_async_copy(dst1.at[pl.ds(0, CH)], dstb.at[slot],
                              isem[slot]).wait()
        pltpu.make_async_copy(w1.at[pl.ds(0, CH)], wb.at[slot],
                              isem[slot]).wait()

    for j in range(5):
        _fire_idx(j, j)
    for b in range(2):
        _wait_idx(b)
        pltpu.async_copy(support.at[srcb.at[b]], gbuf[b], gsem[b])

    def _outer(g, carry):
        for b6 in range(6):
            i = g * 6 + b6
            b3 = b6 % 3

            pltpu.make_async_copy(support.at[srcb.at[b6]], gbuf[b3],
                                  gsem[b3]).wait()

            def _scale_group(grp, cc):
                wg = wb[b6, pl.ds(grp * L, L)]
                base = grp * L
                for t in range(L):
                    wvec = jnp.full((L,), wg[t], jnp.float32)
                    for j in range(D // L):
                        sl = pl.ds(j * L, L)
                        gbuf[b3][base + t, sl] = gbuf[b3][base + t, sl] * wvec
                return cc

            lax.fori_loop(0, CH // L, _scale_group, 0)

            pltpu.sync_copy(gbuf[b3], acc.at[dstb.at[b6]], add=True)

            def _refill_idx():
                _fire_idx(i + 5, (b6 + 5) % 6)

            if b6 == 0:
                _refill_idx()
            else:
                @pl.when(g < gb - 1)
                def _():
                    _refill_idx()

            def _fire_gather():
                _wait_idx((b6 + 2) % 6)
                pltpu.async_copy(support.at[srcb.at[(b6 + 2) % 6]],
                                 gbuf[(b6 + 2) % 3], gsem[(b6 + 2) % 3])

            if b6 <= 3:
                _fire_gather()
            else:
                @pl.when(g < gb - 1)
                def _():
                    _fire_gather()
        return carry

    lax.fori_loop(0, gb, _outer, 0)

    plsc.subcore_barrier()

    pltpu.sync_copy(acc.at[pl.ds(s * RPT, RPT)],
                    out.at[c, pl.ds(s * RPT, RPT)])


def _mm_body(x_ref, w_ref, o_ref):
    o_ref[...] = jnp.dot(x_ref[...], w_ref[...],
                         preferred_element_type=jnp.float32)


def _combine_body(p_ref, b_ref, o_ref):
    o_ref[...] = (p_ref[0, :N_NODES, :] + p_ref[1, :N_NODES, :]
                  + b_ref[...])


def kernel(x, edge_index, edge_weight, W, b):
    support = pl.pallas_call(
        _mm_body,
        out_shape=jax.ShapeDtypeStruct((N_NODES, D), jnp.float32),
    )(x, W)

    pad = E_PAD - E
    src1 = jnp.concatenate(
        [edge_index[1].astype(jnp.int32), jnp.zeros((pad,), jnp.int32)])
    dst1 = jnp.concatenate(
        [edge_index[0].astype(jnp.int32), jnp.zeros((pad,), jnp.int32)])
    w1 = jnp.concatenate(
        [edge_weight, jnp.zeros((pad,), jnp.float32)])

    partials = _spmm(support, src1, dst1, w1)

    return pl.pallas_call(
        _combine_body,
        out_shape=jax.ShapeDtypeStruct((N_NODES, D), jnp.float32),
    )(partials, b)


# speedup vs baseline: 1.3248x; 1.0793x over previous
import functools

import jax
import jax.numpy as jnp
from jax import lax
from jax.experimental import pallas as pl
from jax.experimental.pallas import tpu as pltpu
from jax.experimental.pallas import tpu_sc as plsc

N_NODES = 10000
D = 128
E = 320000
NC = 2
NS = 16
L = 16
NW = NC * NS
CH = 112
NCHUNK = 90
K0 = 132
K1 = 2 * NCHUNK - K0
E_PAD = NW * NCHUNK * CH
N_PAD = 10112
RPT = N_PAD // NS

_mesh = plsc.VectorSubcoreMesh(core_axis_name="c", subcore_axis_name="s")


@functools.partial(
    pl.kernel,
    out_type=jax.ShapeDtypeStruct((NC, N_PAD, D), jnp.float32),
    mesh=_mesh,
    scratch_types=[
        [pltpu.VMEM((CH, D), jnp.float32) for _ in range(3)],
        pltpu.VMEM((6, CH), jnp.int32),
        pltpu.VMEM((6, CH), jnp.int32),
        pltpu.VMEM((6, CH), jnp.float32),
        pltpu.VMEM_SHARED((N_PAD, D), jnp.float32),
        [pltpu.SemaphoreType.DMA for _ in range(3)],
        [pltpu.SemaphoreType.DMA for _ in range(3)],
        [pltpu.SemaphoreType.DMA for _ in range(6)],
    ],
)
def _spmm(support, src1, dst1, w1, out,
          gbuf, srcb, dstb, wb, acc, gsem, ssem, isem):
    c = lax.axis_index("c")
    s = lax.axis_index("s")
    k_chunks = jnp.where(c == 0, K0, K1)
    gb = k_chunks // 6
    chunk_base = jnp.where(c == 0, s * K0, NS * K0 + s * K1)
    edge_base = chunk_base * CH

    zeros = jnp.zeros((L,), jnp.float32)

    def _zero_row(e, carry):
        for j in range(D // L):
            gbuf[2][e, pl.ds(j * L, L)] = zeros
        return carry

    lax.fori_loop(0, CH, _zero_row, 0)
    base_row = s * RPT
    for r in range(RPT // CH):
        pltpu.sync_copy(gbuf[2], acc.at[pl.ds(base_row + r * CH, CH)])
    pltpu.sync_copy(gbuf[2].at[pl.ds(0, RPT % CH)],
                    acc.at[pl.ds(base_row + (RPT // CH) * CH, RPT % CH)])
    plsc.subcore_barrier()

    def _fire_idx(j, slot):
        off = edge_base + j * CH
        pltpu.async_copy(src1.at[pl.ds(off, CH)], srcb.at[slot], isem[slot])
        pltpu.async_copy(dst1.at[pl.ds(off, CH)], dstb.at[slot], isem[slot])
        pltpu.async_copy(w1.at[pl.ds(off, CH)], wb.at[slot], isem[slot])

    def _wait_idx(slot):
        pltpu.make_async_copy(src1.at[pl.ds(0, CH)], srcb.at[slot],
                              isem[slot]).wait()
        pltpu.make_async_copy(dst1.at[pl.ds(0, CH)], dstb.at[slot],
                              isem[slot]).wait()
        pltpu.make_async_copy(w1.at[pl.ds(0, CH)], wb.at[slot],
                              isem[slot]).wait()

    for j in range(5):
        _fire_idx(j, j)
    for b in range(2):
        _wait_idx(b)
        pltpu.async_copy(support.at[srcb.at[b]], gbuf[b], gsem[b])

    def _outer(g, carry):
        for b6 in range(6):
            i = g * 6 + b6
            b3 = b6 % 3

            pltpu.make_async_copy(support.at[srcb.at[b6]], gbuf[b3],
                                  gsem[b3]).wait()

            @functools.partial(plsc.parallel_loop, 0, CH // L, unroll=2)
            def _scale_group(grp):
                wg = wb[b6, pl.ds(grp * L, L)]
                base = grp * L
                for t in range(L):
                    wvec = jnp.full((L,), wg[t], jnp.float32)
                    for j in range(D // L):
                        sl = pl.ds(j * L, L)
                        gbuf[b3][base + t, sl] = gbuf[b3][base + t, sl] * wvec

            pltpu.sync_copy(gbuf[b3], acc.at[dstb.at[b6]], add=True)

            def _refill_idx():
                _fire_idx(i + 5, (b6 + 5) % 6)

            if b6 == 0:
                _refill_idx()
            else:
                @pl.when(g < gb - 1)
                def _():
                    _refill_idx()

            def _fire_gather():
                _wait_idx((b6 + 2) % 6)
                pltpu.async_copy(support.at[srcb.at[(b6 + 2) % 6]],
                                 gbuf[(b6 + 2) % 3], gsem[(b6 + 2) % 3])

            if b6 <= 3:
                _fire_gather()
            else:
                @pl.when(g < gb - 1)
                def _():
                    _fire_gather()
        return carry

    lax.fori_loop(0, gb, _outer, 0)

    plsc.subcore_barrier()

    pltpu.sync_copy(acc.at[pl.ds(s * RPT, RPT)],
                    out.at[c, pl.ds(s * RPT, RPT)])


def _mm_body(x_ref, w_ref, o_ref):
    o_ref[...] = jnp.dot(x_ref[...], w_ref[...],
                         preferred_element_type=jnp.float32)


def _combine_body(p_ref, b_ref, o_ref):
    o_ref[...] = (p_ref[0, :N_NODES, :] + p_ref[1, :N_NODES, :]
                  + b_ref[...])


def kernel(x, edge_index, edge_weight, W, b):
    support = pl.pallas_call(
        _mm_body,
        out_shape=jax.ShapeDtypeStruct((N_NODES, D), jnp.float32),
    )(x, W)

    pad = E_PAD - E
    src1 = jnp.concatenate(
        [edge_index[1].astype(jnp.int32), jnp.zeros((pad,), jnp.int32)])
    dst1 = jnp.concatenate(
        [edge_index[0].astype(jnp.int32), jnp.zeros((pad,), jnp.int32)])
    w1 = jnp.concatenate(
        [edge_weight, jnp.zeros((pad,), jnp.float32)])

    partials = _spmm(support, src1, dst1, w1)

    return pl.pallas_call(
        _combine_body,
        out_shape=jax.ShapeDtypeStruct((N_NODES, D), jnp.float32),
    )(partials, b)
